# baseline (speedup 1.0000x reference)
import functools

import jax
import jax.numpy as jnp
from jax import lax
from jax.experimental import pallas as pl
from jax.experimental.pallas import tpu as pltpu
from jax.experimental.pallas import tpu_sc as plsc

B = 16384
D = 32
NC = 2
NS = 16
NW = NC * NS
BPW = B // NW

_mesh = plsc.VectorSubcoreMesh(core_axis_name="c", subcore_axis_name="s")


@functools.partial(
    pl.kernel,
    mesh=_mesh,
    compiler_params=pltpu.CompilerParams(use_tc_tiling_on_sc=False),
    out_type=[
        jax.ShapeDtypeStruct((B,), jnp.float32),
        jax.ShapeDtypeStruct((B,), jnp.float32),
    ],
    scratch_types=[
        pltpu.VMEM((BPW,), jnp.int32),
        pltpu.VMEM((BPW,), jnp.int32),
        pltpu.VMEM((BPW,), jnp.int32),
        pltpu.VMEM((D, BPW), jnp.float32),
        pltpu.VMEM((D, BPW), jnp.float32),
        pltpu.VMEM((D, BPW), jnp.float32),
        pltpu.VMEM((BPW,), jnp.float32),
        pltpu.VMEM((BPW,), jnp.float32),
        pltpu.SemaphoreType.DMA,
        pltpu.SemaphoreType.DMA,
        pltpu.SemaphoreType.DMA,
    ],
)
def _bpr_sc(user_hbm, item_i_hbm, item_j_hbm, uwT_hbm, iwT_hbm,
            out_i_hbm, out_j_hbm,
            uidx, iidx, jidx, ug, ig, jg, oi, oj,
            su, si, sj):
    wid = lax.axis_index("s") * NC + lax.axis_index("c")
    base = wid * BPW

    pltpu.sync_copy(user_hbm.at[pl.ds(base, BPW)], uidx)
    pltpu.sync_copy(item_i_hbm.at[pl.ds(base, BPW)], iidx)
    pltpu.sync_copy(item_j_hbm.at[pl.ds(base, BPW)], jidx)

    copies = []
    for f in range(D):
        copies.append(pltpu.async_copy(uwT_hbm.at[f].at[uidx], ug.at[f], su))
        copies.append(pltpu.async_copy(iwT_hbm.at[f].at[iidx], ig.at[f], si))
        copies.append(pltpu.async_copy(iwT_hbm.at[f].at[jidx], jg.at[f], sj))
    for cp in copies:
        cp.wait()

    def chunk_body(c, carry):
        acc_i = jnp.zeros((16,), jnp.float32)
        acc_j = jnp.zeros((16,), jnp.float32)
        for f in range(D):
            u = ug[f, pl.ds(c * 16, 16)]
            i = ig[f, pl.ds(c * 16, 16)]
            j = jg[f, pl.ds(c * 16, 16)]
            acc_i = acc_i + u * i
            acc_j = acc_j + u * j
        oi[pl.ds(c * 16, 16)] = acc_i
        oj[pl.ds(c * 16, 16)] = acc_j
        return carry

    lax.fori_loop(0, BPW // 16, chunk_body, 0)

    pltpu.sync_copy(oi, out_i_hbm.at[pl.ds(base, BPW)])
    pltpu.sync_copy(oj, out_j_hbm.at[pl.ds(base, BPW)])


def kernel(user, item_i, item_j, embed_user_weight, embed_item_weight):
    user = user.astype(jnp.int32)
    item_i = item_i.astype(jnp.int32)
    item_j = item_j.astype(jnp.int32)
    uwT = embed_user_weight.T
    iwT = embed_item_weight.T
    pi, pj = _bpr_sc(user, item_i, item_j, uwT, iwT)
    return pi, pj


# speedup vs baseline: 18.8453x; 18.8453x over previous
import functools

import jax
import jax.numpy as jnp
from jax import lax
from jax.experimental import pallas as pl
from jax.experimental.pallas import tpu as pltpu
from jax.experimental.pallas import tpu_sc as plsc

B = 16384
D = 32
V = 1000000
NC = 2
NS = 16
NW = NC * NS
BPW = B // NW
VMAIN = (V // 128) * 128
VTAIL = V - VMAIN
WCH = 27776
NCHD = VMAIN // WCH

_mesh = plsc.VectorSubcoreMesh(core_axis_name="c", subcore_axis_name="s")


@functools.partial(
    pl.kernel,
    mesh=_mesh,
    out_type=[
        jax.ShapeDtypeStruct((D * V,), jnp.float32),
        jax.ShapeDtypeStruct((D * V,), jnp.float32),
    ],
    scratch_types=[
        pltpu.VMEM((WCH,), jnp.float32),
        pltpu.VMEM((WCH,), jnp.float32),
        pltpu.VMEM((D * VTAIL,), jnp.float32),
        pltpu.VMEM((D * VTAIL,), jnp.float32),
        pltpu.SemaphoreType.DMA,
        pltpu.SemaphoreType.DMA,
        pltpu.SemaphoreType.DMA,
        pltpu.SemaphoreType.DMA,
    ],
)
def _detile(uwT_hbm, iwT_hbm, tail_u_hbm, tail_i_hbm, out_u_hbm, out_i_hbm,
            buf0, buf1, tbu, tbi, r0, r1, w0, w1):
    wid = lax.axis_index("s") * NC + lax.axis_index("c")
    bufs = (buf0, buf1)
    rsems = (r0, r1)
    wsems = (w0, w1)

    f = wid
    for src_hbm, dst_hbm, tail_hbm, tbuf in (
        (uwT_hbm, out_u_hbm, tail_u_hbm, tbu),
        (iwT_hbm, out_i_hbm, tail_i_hbm, tbi),
    ):
        pltpu.sync_copy(tail_hbm, tbuf)
        pltpu.sync_copy(tbuf.at[pl.ds(f * VTAIL, VTAIL)],
                        dst_hbm.at[pl.ds(f * V + VMAIN, VTAIL)])
        writes = [None, None]
        reads = [None, None]
        for c in range(NCHD):
            slot = c % 2
            if writes[slot] is not None:
                writes[slot].wait()
            start = c * WCH
            reads[slot] = pltpu.async_copy(
                src_hbm.at[f, pl.ds(start, WCH)], bufs[slot], rsems[slot])
            reads[slot].wait()
            writes[slot] = pltpu.async_copy(
                bufs[slot], dst_hbm.at[pl.ds(f * V + start, WCH)],
                wsems[slot])
        for w in writes:
            if w is not None:
                w.wait()


@functools.partial(
    pl.kernel,
    mesh=_mesh,
    compiler_params=pltpu.CompilerParams(use_tc_tiling_on_sc=False),
    out_type=[
        jax.ShapeDtypeStruct((B,), jnp.float32),
        jax.ShapeDtypeStruct((B,), jnp.float32),
    ],
    scratch_types=[
        pltpu.VMEM((BPW,), jnp.int32),
        pltpu.VMEM((BPW,), jnp.int32),
        pltpu.VMEM((BPW,), jnp.int32),
        pltpu.VMEM((D, BPW), jnp.float32),
        pltpu.VMEM((D, BPW), jnp.float32),
        pltpu.VMEM((D, BPW), jnp.float32),
        pltpu.VMEM((BPW,), jnp.float32),
        pltpu.VMEM((BPW,), jnp.float32),
        pltpu.SemaphoreType.DMA,
        pltpu.SemaphoreType.DMA,
        pltpu.SemaphoreType.DMA,
    ],
)
def _bpr_gather(user_hbm, item_i_hbm, item_j_hbm, uwT_hbm, iwT_hbm,
                out_i_hbm, out_j_hbm,
                uidx, iidx, jidx, ug, ig, jg, oi, oj,
                su, si, sj):
    wid = lax.axis_index("s") * NC + lax.axis_index("c")
    base = wid * BPW

    pltpu.sync_copy(user_hbm.at[pl.ds(base, BPW)], uidx)
    pltpu.sync_copy(item_i_hbm.at[pl.ds(base, BPW)], iidx)
    pltpu.sync_copy(item_j_hbm.at[pl.ds(base, BPW)], jidx)

    copies = []
    for f in range(D):
        copies.append(pltpu.async_copy(uwT_hbm.at[f].at[uidx], ug.at[f], su))
        copies.append(pltpu.async_copy(iwT_hbm.at[f].at[iidx], ig.at[f], si))
        copies.append(pltpu.async_copy(iwT_hbm.at[f].at[jidx], jg.at[f], sj))
    for cp in copies:
        cp.wait()

    def chunk_body(c, carry):
        sl = pl.ds(c * 16, 16)
        acc_i = jnp.zeros((16,), jnp.float32)
        acc_j = jnp.zeros((16,), jnp.float32)
        for f in range(D):
            u = ug[f, sl]
            i = ig[f, sl]
            j = jg[f, sl]
            acc_i = acc_i + u * i
            acc_j = acc_j + u * j
        oi[sl] = acc_i
        oj[sl] = acc_j
        return carry

    lax.fori_loop(0, BPW // 16, chunk_body, 0)

    pltpu.sync_copy(oi, out_i_hbm.at[pl.ds(base, BPW)])
    pltpu.sync_copy(oj, out_j_hbm.at[pl.ds(base, BPW)])


def kernel(user, item_i, item_j, embed_user_weight, embed_item_weight):
    user = user.astype(jnp.int32)
    item_i = item_i.astype(jnp.int32)
    item_j = item_j.astype(jnp.int32)
    uwT = embed_user_weight.T
    iwT = embed_item_weight.T
    tail_u = lax.slice(embed_user_weight, (VMAIN, 0), (V, D)).T.reshape(-1)
    tail_i = lax.slice(embed_item_weight, (VMAIN, 0), (V, D)).T.reshape(-1)
    flat_u, flat_i = _detile(uwT, iwT, tail_u, tail_i)
    pi, pj = _bpr_gather(user, item_i, item_j,
                         flat_u.reshape(D, V), flat_i.reshape(D, V))
    return pi, pj


# speedup vs baseline: 19.1068x; 1.0139x over previous
import functools

import jax
import jax.numpy as jnp
from jax import lax
from jax.experimental import pallas as pl
from jax.experimental.pallas import tpu as pltpu
from jax.experimental.pallas import tpu_sc as plsc

B = 16384
D = 32
V = 1000000
NC = 2
NS = 16
NW = NC * NS
BPW = B // NW
VMAIN = (V // 128) * 128
VTAIL = V - VMAIN
WCH = 27776
NCHD = VMAIN // WCH

_mesh = plsc.VectorSubcoreMesh(core_axis_name="c", subcore_axis_name="s")


@functools.partial(
    pl.kernel,
    mesh=_mesh,
    out_type=[
        jax.ShapeDtypeStruct((D * V,), jnp.float32),
        jax.ShapeDtypeStruct((D * V,), jnp.float32),
    ],
    scratch_types=[
        pltpu.VMEM((WCH,), jnp.float32),
        pltpu.VMEM((WCH,), jnp.float32),
        pltpu.VMEM((WCH,), jnp.float32),
        pltpu.VMEM((WCH,), jnp.float32),
        pltpu.VMEM((D * VTAIL,), jnp.float32),
        pltpu.VMEM((D * VTAIL,), jnp.float32),
        pltpu.SemaphoreType.DMA,
        pltpu.SemaphoreType.DMA,
        pltpu.SemaphoreType.DMA,
        pltpu.SemaphoreType.DMA,
        pltpu.SemaphoreType.DMA,
        pltpu.SemaphoreType.DMA,
        pltpu.SemaphoreType.DMA,
        pltpu.SemaphoreType.DMA,
    ],
)
def _detile(uwT_hbm, iwT_hbm, tail_u_hbm, tail_i_hbm, out_u_hbm, out_i_hbm,
            buf0, buf1, buf2, buf3, tbu, tbi,
            r0, r1, r2, r3, w0, w1, w2, w3):
    wid = lax.axis_index("s") * NC + lax.axis_index("c")
    bufs = (buf0, buf1, buf2, buf3)
    rsems = (r0, r1, r2, r3)
    wsems = (w0, w1, w2, w3)
    LOOKAHEAD = 2

    f = wid
    for src_hbm, dst_hbm, tail_hbm, tbuf in (
        (uwT_hbm, out_u_hbm, tail_u_hbm, tbu),
        (iwT_hbm, out_i_hbm, tail_i_hbm, tbi),
    ):
        pltpu.sync_copy(tail_hbm, tbuf)
        pltpu.sync_copy(tbuf.at[pl.ds(f * VTAIL, VTAIL)],
                        dst_hbm.at[pl.ds(f * V + VMAIN, VTAIL)])
        writes = [None, None, None, None]
        reads = [None, None, None, None]
        for c in range(NCHD + LOOKAHEAD):
            if c < NCHD:
                slot = c % 4
                if writes[slot] is not None:
                    writes[slot].wait()
                reads[slot] = pltpu.async_copy(
                    src_hbm.at[f, pl.ds(c * WCH, WCH)], bufs[slot],
                    rsems[slot])
            if c >= LOOKAHEAD:
                cc = c - LOOKAHEAD
                slot = cc % 4
                reads[slot].wait()
                writes[slot] = pltpu.async_copy(
                    bufs[slot], dst_hbm.at[pl.ds(f * V + cc * WCH, WCH)],
                    wsems[slot])
        for w in writes:
            if w is not None:
                w.wait()


@functools.partial(
    pl.kernel,
    mesh=_mesh,
    compiler_params=pltpu.CompilerParams(use_tc_tiling_on_sc=False),
    out_type=[
        jax.ShapeDtypeStruct((B,), jnp.float32),
        jax.ShapeDtypeStruct((B,), jnp.float32),
    ],
    scratch_types=[
        pltpu.VMEM((BPW,), jnp.int32),
        pltpu.VMEM((BPW,), jnp.int32),
        pltpu.VMEM((BPW,), jnp.int32),
        pltpu.VMEM((D, BPW), jnp.float32),
        pltpu.VMEM((D, BPW), jnp.float32),
        pltpu.VMEM((D, BPW), jnp.float32),
        pltpu.VMEM((BPW,), jnp.float32),
        pltpu.VMEM((BPW,), jnp.float32),
        pltpu.SemaphoreType.DMA,
        pltpu.SemaphoreType.DMA,
        pltpu.SemaphoreType.DMA,
    ],
)
def _bpr_gather(user_hbm, item_i_hbm, item_j_hbm, uwT_hbm, iwT_hbm,
                out_i_hbm, out_j_hbm,
                uidx, iidx, jidx, ug, ig, jg, oi, oj,
                su, si, sj):
    wid = lax.axis_index("s") * NC + lax.axis_index("c")
    base = wid * BPW

    pltpu.sync_copy(user_hbm.at[pl.ds(base, BPW)], uidx)
    pltpu.sync_copy(item_i_hbm.at[pl.ds(base, BPW)], iidx)
    pltpu.sync_copy(item_j_hbm.at[pl.ds(base, BPW)], jidx)

    copies = []
    for f in range(D):
        copies.append(pltpu.async_copy(uwT_hbm.at[f].at[uidx], ug.at[f], su))
        copies.append(pltpu.async_copy(iwT_hbm.at[f].at[iidx], ig.at[f], si))
        copies.append(pltpu.async_copy(iwT_hbm.at[f].at[jidx], jg.at[f], sj))
    for cp in copies:
        cp.wait()

    def chunk_body(c, carry):
        sl = pl.ds(c * 16, 16)
        acc_i = jnp.zeros((16,), jnp.float32)
        acc_j = jnp.zeros((16,), jnp.float32)
        for f in range(D):
            u = ug[f, sl]
            i = ig[f, sl]
            j = jg[f, sl]
            acc_i = acc_i + u * i
            acc_j = acc_j + u * j
        oi[sl] = acc_i
        oj[sl] = acc_j
        return carry

    lax.fori_loop(0, BPW // 16, chunk_body, 0)

    pltpu.sync_copy(oi, out_i_hbm.at[pl.ds(base, BPW)])
    pltpu.sync_copy(oj, out_j_hbm.at[pl.ds(base, BPW)])


def kernel(user, item_i, item_j, embed_user_weight, embed_item_weight):
    user = user.astype(jnp.int32)
    item_i = item_i.astype(jnp.int32)
    item_j = item_j.astype(jnp.int32)
    uwT = embed_user_weight.T
    iwT = embed_item_weight.T
    tail_u = lax.slice(embed_user_weight, (VMAIN, 0), (V, D)).T.reshape(-1)
    tail_i = lax.slice(embed_item_weight, (VMAIN, 0), (V, D)).T.reshape(-1)
    flat_u, flat_i = _detile(uwT, iwT, tail_u, tail_i)
    pi, pj = _bpr_gather(user, item_i, item_j,
                         flat_u.reshape(D, V), flat_i.reshape(D, V))
    return pi, pj


# speedup vs baseline: 19.2979x; 1.0100x over previous
import functools

import jax
import jax.numpy as jnp
from jax import lax
from jax.experimental import pallas as pl
from jax.experimental.pallas import tpu as pltpu
from jax.experimental.pallas import tpu_sc as plsc

B = 16384
D = 32
V = 1000000
NC = 2
NS = 16
NW = NC * NS
BPW = B // NW
VMAIN = (V // 128) * 128
VTAIL = V - VMAIN
WCH = 27776
NCHD = VMAIN // WCH

_mesh = plsc.VectorSubcoreMesh(core_axis_name="c", subcore_axis_name="s")


@functools.partial(
    pl.kernel,
    mesh=_mesh,
    out_type=[
        jax.ShapeDtypeStruct((D * V,), jnp.float32),
        jax.ShapeDtypeStruct((D * V,), jnp.float32),
    ],
    scratch_types=[
        pltpu.VMEM((WCH,), jnp.float32),
        pltpu.VMEM((WCH,), jnp.float32),
        pltpu.VMEM((WCH,), jnp.float32),
        pltpu.VMEM((WCH,), jnp.float32),
        pltpu.VMEM((D * VTAIL,), jnp.float32),
        pltpu.VMEM((D * VTAIL,), jnp.float32),
        pltpu.SemaphoreType.DMA,
        pltpu.SemaphoreType.DMA,
        pltpu.SemaphoreType.DMA,
        pltpu.SemaphoreType.DMA,
        pltpu.SemaphoreType.DMA,
        pltpu.SemaphoreType.DMA,
        pltpu.SemaphoreType.DMA,
        pltpu.SemaphoreType.DMA,
    ],
)
def _detile(uwT_hbm, iwT_hbm, tail_u_hbm, tail_i_hbm, out_u_hbm, out_i_hbm,
            buf0, buf1, buf2, buf3, tbu, tbi,
            r0, r1, r2, r3, w0, w1, w2, w3):
    wid = lax.axis_index("s") * NC + lax.axis_index("c")
    bufs = (buf0, buf1, buf2, buf3)
    rsems = (r0, r1, r2, r3)
    wsems = (w0, w1, w2, w3)
    LOOKAHEAD = 3

    f = wid
    for src_hbm, dst_hbm, tail_hbm, tbuf in (
        (uwT_hbm, out_u_hbm, tail_u_hbm, tbu),
        (iwT_hbm, out_i_hbm, tail_i_hbm, tbi),
    ):
        pltpu.sync_copy(tail_hbm, tbuf)
        pltpu.sync_copy(tbuf.at[pl.ds(f * VTAIL, VTAIL)],
                        dst_hbm.at[pl.ds(f * V + VMAIN, VTAIL)])
        writes = [None, None, None, None]
        reads = [None, None, None, None]
        for c in range(NCHD + LOOKAHEAD):
            if c < NCHD:
                slot = c % 4
                if writes[slot] is not None:
                    writes[slot].wait()
                reads[slot] = pltpu.async_copy(
                    src_hbm.at[f, pl.ds(c * WCH, WCH)], bufs[slot],
                    rsems[slot])
            if c >= LOOKAHEAD:
                cc = c - LOOKAHEAD
                slot = cc % 4
                reads[slot].wait()
                writes[slot] = pltpu.async_copy(
                    bufs[slot], dst_hbm.at[pl.ds(f * V + cc * WCH, WCH)],
                    wsems[slot])
        for w in writes:
            if w is not None:
                w.wait()


@functools.partial(
    pl.kernel,
    mesh=_mesh,
    compiler_params=pltpu.CompilerParams(use_tc_tiling_on_sc=False),
    out_type=[
        jax.ShapeDtypeStruct((B,), jnp.float32),
        jax.ShapeDtypeStruct((B,), jnp.float32),
    ],
    scratch_types=[
        pltpu.VMEM((BPW,), jnp.int32),
        pltpu.VMEM((BPW,), jnp.int32),
        pltpu.VMEM((BPW,), jnp.int32),
        pltpu.VMEM((D, BPW), jnp.float32),
        pltpu.VMEM((D, BPW), jnp.float32),
        pltpu.VMEM((D, BPW), jnp.float32),
        pltpu.VMEM((BPW,), jnp.float32),
        pltpu.VMEM((BPW,), jnp.float32),
        pltpu.SemaphoreType.DMA,
        pltpu.SemaphoreType.DMA,
        pltpu.SemaphoreType.DMA,
    ],
)
def _bpr_gather(user_hbm, item_i_hbm, item_j_hbm, uwT_hbm, iwT_hbm,
                out_i_hbm, out_j_hbm,
                uidx, iidx, jidx, ug, ig, jg, oi, oj,
                su, si, sj):
    wid = lax.axis_index("s") * NC + lax.axis_index("c")
    base = wid * BPW

    pltpu.sync_copy(user_hbm.at[pl.ds(base, BPW)], uidx)
    pltpu.sync_copy(item_i_hbm.at[pl.ds(base, BPW)], iidx)
    pltpu.sync_copy(item_j_hbm.at[pl.ds(base, BPW)], jidx)

    copies = []
    for f in range(D):
        copies.append(pltpu.async_copy(uwT_hbm.at[f].at[uidx], ug.at[f], su))
        copies.append(pltpu.async_copy(iwT_hbm.at[f].at[iidx], ig.at[f], si))
        copies.append(pltpu.async_copy(iwT_hbm.at[f].at[jidx], jg.at[f], sj))
    for cp in copies:
        cp.wait()

    def chunk_body(c, carry):
        sl = pl.ds(c * 16, 16)
        acc_i = jnp.zeros((16,), jnp.float32)
        acc_j = jnp.zeros((16,), jnp.float32)
        for f in range(D):
            u = ug[f, sl]
            i = ig[f, sl]
            j = jg[f, sl]
            acc_i = acc_i + u * i
            acc_j = acc_j + u * j
        oi[sl] = acc_i
        oj[sl] = acc_j
        return carry

    lax.fori_loop(0, BPW // 16, chunk_body, 0)

    pltpu.sync_copy(oi, out_i_hbm.at[pl.ds(base, BPW)])
    pltpu.sync_copy(oj, out_j_hbm.at[pl.ds(base, BPW)])


def kernel(user, item_i, item_j, embed_user_weight, embed_item_weight):
    user = user.astype(jnp.int32)
    item_i = item_i.astype(jnp.int32)
    item_j = item_j.astype(jnp.int32)
    uwT = embed_user_weight.T
    iwT = embed_item_weight.T
    tail_u = lax.slice(embed_user_weight, (VMAIN, 0), (V, D)).T.reshape(-1)
    tail_i = lax.slice(embed_item_weight, (VMAIN, 0), (V, D)).T.reshape(-1)
    flat_u, flat_i = _detile(uwT, iwT, tail_u, tail_i)
    pi, pj = _bpr_gather(user, item_i, item_j,
                         flat_u.reshape(D, V), flat_i.reshape(D, V))
    return pi, pj


# speedup vs baseline: 19.3912x; 1.0048x over previous
import functools

import jax
import jax.numpy as jnp
from jax import lax
from jax.experimental import pallas as pl
from jax.experimental.pallas import tpu as pltpu
from jax.experimental.pallas import tpu_sc as plsc

B = 16384
D = 32
V = 1000000
NC = 2
NS = 16
NW = NC * NS
BPW = B // NW
VMAIN = (V // 128) * 128
VTAIL = V - VMAIN
WCH = 27776
NCHD = VMAIN // WCH

_mesh = plsc.VectorSubcoreMesh(core_axis_name="c", subcore_axis_name="s")


@functools.partial(
    pl.kernel,
    mesh=_mesh,
    out_type=[
        jax.ShapeDtypeStruct((D * V,), jnp.float32),
        jax.ShapeDtypeStruct((D * V,), jnp.float32),
    ],
    scratch_types=[
        pltpu.VMEM((WCH,), jnp.float32),
        pltpu.VMEM((WCH,), jnp.float32),
        pltpu.VMEM((WCH,), jnp.float32),
        pltpu.VMEM((WCH,), jnp.float32),
        pltpu.VMEM((D * VTAIL,), jnp.float32),
        pltpu.VMEM((D * VTAIL,), jnp.float32),
        pltpu.SemaphoreType.DMA,
        pltpu.SemaphoreType.DMA,
        pltpu.SemaphoreType.DMA,
        pltpu.SemaphoreType.DMA,
        pltpu.SemaphoreType.DMA,
        pltpu.SemaphoreType.DMA,
        pltpu.SemaphoreType.DMA,
        pltpu.SemaphoreType.DMA,
    ],
)
def _detile(uwT_hbm, iwT_hbm, tail_u_hbm, tail_i_hbm, out_u_hbm, out_i_hbm,
            buf0, buf1, buf2, buf3, tbu, tbi,
            r0, r1, r2, r3, w0, w1, w2, w3):
    wid = lax.axis_index("s") * NC + lax.axis_index("c")
    bufs = (buf0, buf1, buf2, buf3)
    rsems = (r0, r1, r2, r3)
    wsems = (w0, w1, w2, w3)
    LOOKAHEAD = 2

    f = wid
    for src_hbm, dst_hbm, tail_hbm, tbuf in (
        (uwT_hbm, out_u_hbm, tail_u_hbm, tbu),
        (iwT_hbm, out_i_hbm, tail_i_hbm, tbi),
    ):
        pltpu.sync_copy(tail_hbm, tbuf)
        pltpu.sync_copy(tbuf.at[pl.ds(f * VTAIL, VTAIL)],
                        dst_hbm.at[pl.ds(f * V + VMAIN, VTAIL)])
        writes = [None, None, None, None]
        reads = [None, None, None, None]
        for c in range(NCHD + LOOKAHEAD):
            if c < NCHD:
                slot = c % 4
                if writes[slot] is not None:
                    writes[slot].wait()
                reads[slot] = pltpu.async_copy(
                    src_hbm.at[f, pl.ds(c * WCH, WCH)], bufs[slot],
                    rsems[slot])
            if c >= LOOKAHEAD:
                cc = c - LOOKAHEAD
                slot = cc % 4
                reads[slot].wait()
                writes[slot] = pltpu.async_copy(
                    bufs[slot], dst_hbm.at[pl.ds(f * V + cc * WCH, WCH)],
                    wsems[slot])
        for w in writes:
            if w is not None:
                w.wait()


@functools.partial(
    pl.kernel,
    mesh=_mesh,
    compiler_params=pltpu.CompilerParams(use_tc_tiling_on_sc=False),
    out_type=[
        jax.ShapeDtypeStruct((B,), jnp.float32),
        jax.ShapeDtypeStruct((B,), jnp.float32),
    ],
    scratch_types=[
        pltpu.VMEM((BPW,), jnp.int32),
        pltpu.VMEM((BPW,), jnp.int32),
        pltpu.VMEM((BPW,), jnp.int32),
        pltpu.VMEM((D, BPW), jnp.float32),
        pltpu.VMEM((D, BPW), jnp.float32),
        pltpu.VMEM((D, BPW), jnp.float32),
        pltpu.VMEM((BPW,), jnp.float32),
        pltpu.VMEM((BPW,), jnp.float32),
        pltpu.SemaphoreType.DMA,
        pltpu.SemaphoreType.DMA,
        pltpu.SemaphoreType.DMA,
    ],
)
def _bpr_gather(user_hbm, item_i_hbm, item_j_hbm, uwT_hbm, iwT_hbm,
                out_i_hbm, out_j_hbm,
                uidx, iidx, jidx, ug, ig, jg, oi, oj,
                su, si, sj):
    wid = lax.axis_index("s") * NC + lax.axis_index("c")
    base = wid * BPW

    pltpu.sync_copy(user_hbm.at[pl.ds(base, BPW)], uidx)
    pltpu.sync_copy(item_i_hbm.at[pl.ds(base, BPW)], iidx)
    pltpu.sync_copy(item_j_hbm.at[pl.ds(base, BPW)], jidx)

    copies = []
    for f in range(D):
        copies.append(pltpu.async_copy(uwT_hbm.at[f].at[uidx], ug.at[f], su))
        copies.append(pltpu.async_copy(iwT_hbm.at[f].at[iidx], ig.at[f], si))
        copies.append(pltpu.async_copy(iwT_hbm.at[f].at[jidx], jg.at[f], sj))
    for cp in copies:
        cp.wait()

    def chunk_body(c, carry):
        sl = pl.ds(c * 16, 16)
        acc_i = jnp.zeros((16,), jnp.float32)
        acc_j = jnp.zeros((16,), jnp.float32)
        for f in range(D):
            u = ug[f, sl]
            i = ig[f, sl]
            j = jg[f, sl]
            acc_i = acc_i + u * i
            acc_j = acc_j + u * j
        oi[sl] = acc_i
        oj[sl] = acc_j
        return carry

    lax.fori_loop(0, BPW // 16, chunk_body, 0)

    pltpu.sync_copy(oi, out_i_hbm.at[pl.ds(base, BPW)])
    pltpu.sync_copy(oj, out_j_hbm.at[pl.ds(base, BPW)])


def kernel(user, item_i, item_j, embed_user_weight, embed_item_weight):
    user = user.astype(jnp.int32)
    item_i = item_i.astype(jnp.int32)
    item_j = item_j.astype(jnp.int32)
    uwT = embed_user_weight.T
    iwT = embed_item_weight.T
    tail_u = lax.slice(embed_user_weight, (VMAIN, 0), (V, D)).T.reshape(-1)
    tail_i = lax.slice(embed_item_weight, (VMAIN, 0), (V, D)).T.reshape(-1)
    flat_u, flat_i = _detile(uwT, iwT, tail_u, tail_i)
    pi, pj = _bpr_gather(user, item_i, item_j,
                         flat_u.reshape(D, V), flat_i.reshape(D, V))
    return pi, pj
